# baseline (speedup 1.0000x reference)
import functools

import jax
import jax.numpy as jnp
from jax import lax
from jax.experimental import pallas as pl
from jax.experimental.pallas import tpu as pltpu
from jax.experimental.pallas import tpu_sc as plsc

NB = 64
NT = 8192
CH = 128
L = 16
NC = 2
NS = 16
NW = NC * NS
SPLIT = NT // 2
TPW = (NT - SPLIT) // NW


def _sc_body(feat_hbm, maskt_hbm, psum_hbm, feat_v, mask_v, acc_v, sem):
    c = lax.axis_index("c")
    s = lax.axis_index("s")
    wid = s * NC + c
    base = wid * TPW

    cp_f = pltpu.async_copy(feat_hbm.at[pl.ds(SPLIT + base, TPW)], feat_v, sem)
    cp_m = pltpu.async_copy(maskt_hbm.at[pl.ds(base, TPW)], mask_v, sem)
    cp_f.wait()
    cp_m.wait()

    zeros = jnp.zeros((L,), jnp.float32)

    def zero_body(b, _):
        for k in range(CH // L):
            acc_v[b, pl.ds(k * L, L)] = zeros
        return ()

    lax.fori_loop(0, NB, zero_body, ())

    def tok_body(tp, _):
        t0 = 2 * tp
        t1 = t0 + 1
        f0 = [feat_v[t0, pl.ds(k * L, L)] for k in range(CH // L)]
        f1 = [feat_v[t1, pl.ds(k * L, L)] for k in range(CH // L)]
        f01 = [f0[k] + f1[k] for k in range(CH // L)]
        for j in range(NB // L):
            sv = mask_v[t0, pl.ds(j * L, L)] + 2.0 * mask_v[t1, pl.ds(j * L, L)]
            for i in range(L):
                b = j * L + i
                s = sv[i]

                @pl.when(s == 3.0)
                def _():
                    for k in range(CH // L):
                        plsc.addupdate(acc_v.at[b, pl.ds(k * L, L)], f01[k])

                @pl.when(s == 1.0)
                def _():
                    for k in range(CH // L):
                        plsc.addupdate(acc_v.at[b, pl.ds(k * L, L)], f0[k])

                @pl.when(s == 2.0)
                def _():
                    for k in range(CH // L):
                        plsc.addupdate(acc_v.at[b, pl.ds(k * L, L)], f1[k])
        return ()

    lax.fori_loop(0, TPW // 2, tok_body, ())

    pltpu.sync_copy(acc_v, psum_hbm.at[wid])


_sc_partial = functools.partial(
    pl.kernel,
    out_type=jax.ShapeDtypeStruct((NW, NB, CH), jnp.float32),
    mesh=plsc.VectorSubcoreMesh(core_axis_name="c", subcore_axis_name="s"),
    scratch_types=[
        pltpu.VMEM((TPW, CH), jnp.float32),
        pltpu.VMEM((TPW, NB), jnp.float32),
        pltpu.VMEM((NB, CH), jnp.float32),
        pltpu.SemaphoreType.DMA,
    ],
)(_sc_body)


def _tc_dense_body(mask_ref, feat_lo_ref, sums_ref, cnts_ref):
    m = mask_ref[...].astype(jnp.float32)
    sums_ref[...] = jax.lax.dot(
        m[:, :SPLIT], feat_lo_ref[...],
        precision=jax.lax.Precision.HIGHEST,
    )
    cnts_ref[...] = jnp.sum(m, axis=1, keepdims=True)


_tc_dense = pl.pallas_call(
    _tc_dense_body,
    out_shape=(
        jax.ShapeDtypeStruct((NB, CH), jnp.float32),
        jax.ShapeDtypeStruct((NB, 1), jnp.float32),
    ),
)


def _combine_body(sums_ref, cnts_ref, psum_ref, out_ref):
    total = sums_ref[...] + jnp.sum(psum_ref[...], axis=0)
    out_ref[...] = total / jnp.maximum(cnts_ref[...], 1.0)


_combine = pl.pallas_call(
    _combine_body,
    out_shape=jax.ShapeDtypeStruct((NB, CH), jnp.float32),
)


def kernel(features, coords, is_inside, bbox_sample_count, batch_splits):
    del coords, bbox_sample_count, batch_splits
    mask32 = is_inside.astype(jnp.int32)
    maskt_hi = is_inside[:, SPLIT:].T.astype(jnp.float32)
    psum = _sc_partial(features, maskt_hi)
    sums_lo, cnts = _tc_dense(mask32, features[:SPLIT])
    return _combine(sums_lo, cnts, psum)


# speedup vs baseline: 2.6680x; 2.6680x over previous
import functools

import jax
import jax.numpy as jnp
from jax import lax
from jax.experimental import pallas as pl
from jax.experimental.pallas import tpu as pltpu
from jax.experimental.pallas import tpu_sc as plsc

NB = 64
NT = 8192
CH = 128
L = 16
NC = 2
NS = 16
NW = NC * NS
SPLIT = 5 * NT // 8
TPW = (NT - SPLIT) // NW


def _sc_body(feat_hbm, maskt_hbm, psum_hbm, feat_v, mask_v, acc_v, sem):
    c = lax.axis_index("c")
    s = lax.axis_index("s")
    wid = s * NC + c
    base = wid * TPW

    cp_f = pltpu.async_copy(feat_hbm.at[pl.ds(SPLIT + base, TPW)], feat_v, sem)
    cp_m = pltpu.async_copy(maskt_hbm.at[pl.ds(base, TPW)], mask_v, sem)
    cp_f.wait()
    cp_m.wait()

    zeros = jnp.zeros((L,), jnp.float32)

    def zero_body(b, _):
        for k in range(CH // L):
            acc_v[b, pl.ds(k * L, L)] = zeros
        return ()

    lax.fori_loop(0, NB, zero_body, ())

    def tok_body(t, _):
        f = [feat_v[t, pl.ds(k * L, L)] for k in range(CH // L)]
        for j in range(NB // L):
            mv = mask_v[t, pl.ds(j * L, L)]
            for i in range(L):
                b = j * L + i

                @pl.when(mv[i] != 0.0)
                def _():
                    for k in range(CH // L):
                        plsc.addupdate(acc_v.at[b, pl.ds(k * L, L)], f[k])
        return ()

    lax.fori_loop(0, TPW, tok_body, ())

    pltpu.sync_copy(acc_v, psum_hbm.at[wid])


_sc_partial = functools.partial(
    pl.kernel,
    out_type=jax.ShapeDtypeStruct((NW, NB, CH), jnp.float32),
    mesh=plsc.VectorSubcoreMesh(core_axis_name="c", subcore_axis_name="s"),
    scratch_types=[
        pltpu.VMEM((TPW, CH), jnp.float32),
        pltpu.VMEM((TPW, NB), jnp.float32),
        pltpu.VMEM((NB, CH), jnp.float32),
        pltpu.SemaphoreType.DMA,
    ],
)(_sc_body)


def _tc_dense_body(mask_ref, feat_lo_ref, sums_ref, cnts_ref):
    m = mask_ref[...].astype(jnp.float32)
    sums_ref[...] = jax.lax.dot(
        m[:, :SPLIT], feat_lo_ref[...],
        precision=jax.lax.Precision.HIGHEST,
    )
    cnts_ref[...] = jnp.sum(m, axis=1, keepdims=True)


_tc_dense = pl.pallas_call(
    _tc_dense_body,
    out_shape=(
        jax.ShapeDtypeStruct((NB, CH), jnp.float32),
        jax.ShapeDtypeStruct((NB, 1), jnp.float32),
    ),
)


def _combine_body(sums_ref, cnts_ref, psum_ref, out_ref):
    total = sums_ref[...] + jnp.sum(psum_ref[...], axis=0)
    out_ref[...] = total / jnp.maximum(cnts_ref[...], 1.0)


_combine = pl.pallas_call(
    _combine_body,
    out_shape=jax.ShapeDtypeStruct((NB, CH), jnp.float32),
)


def kernel(features, coords, is_inside, bbox_sample_count, batch_splits):
    del coords, bbox_sample_count, batch_splits
    mask32 = is_inside.astype(jnp.int32)
    maskt_hi = is_inside[:, SPLIT:].T.astype(jnp.float32)
    psum = _sc_partial(features, maskt_hi)
    sums_lo, cnts = _tc_dense(mask32, features[:SPLIT])
    return _combine(sums_lo, cnts, psum)


# speedup vs baseline: 3.3110x; 1.2410x over previous
import functools

import jax
import jax.numpy as jnp
from jax import lax
from jax.experimental import pallas as pl
from jax.experimental.pallas import tpu as pltpu
from jax.experimental.pallas import tpu_sc as plsc

NB = 64
NT = 8192
CH = 128
L = 16
NC = 2
NS = 16
NW = NC * NS
SPLIT = 3 * NT // 4
TPW = (NT - SPLIT) // NW


def _sc_body(feat_hbm, maskt_hbm, psum_hbm, feat_v, mask_v, acc_v, sem):
    c = lax.axis_index("c")
    s = lax.axis_index("s")
    wid = s * NC + c
    base = wid * TPW

    cp_f = pltpu.async_copy(feat_hbm.at[pl.ds(SPLIT + base, TPW)], feat_v, sem)
    cp_m = pltpu.async_copy(maskt_hbm.at[pl.ds(base, TPW)], mask_v, sem)
    cp_f.wait()
    cp_m.wait()

    zeros = jnp.zeros((L,), jnp.float32)

    def zero_body(b, _):
        for k in range(CH // L):
            acc_v[b, pl.ds(k * L, L)] = zeros
        return ()

    lax.fori_loop(0, NB, zero_body, ())

    def tok_body(t, _):
        f = [feat_v[t, pl.ds(k * L, L)] for k in range(CH // L)]
        for j in range(NB // L):
            mv = mask_v[t, pl.ds(j * L, L)]
            for i in range(L):
                b = j * L + i

                @pl.when(mv[i] != 0.0)
                def _():
                    for k in range(CH // L):
                        plsc.addupdate(acc_v.at[b, pl.ds(k * L, L)], f[k])
        return ()

    lax.fori_loop(0, TPW, tok_body, ())

    pltpu.sync_copy(acc_v, psum_hbm.at[wid])


_sc_partial = functools.partial(
    pl.kernel,
    out_type=jax.ShapeDtypeStruct((NW, NB, CH), jnp.float32),
    mesh=plsc.VectorSubcoreMesh(core_axis_name="c", subcore_axis_name="s"),
    scratch_types=[
        pltpu.VMEM((TPW, CH), jnp.float32),
        pltpu.VMEM((TPW, NB), jnp.float32),
        pltpu.VMEM((NB, CH), jnp.float32),
        pltpu.SemaphoreType.DMA,
    ],
)(_sc_body)


def _tc_dense_body(mask_ref, feat_lo_ref, sums_ref, cnts_ref):
    m = mask_ref[...].astype(jnp.float32)
    sums_ref[...] = jax.lax.dot(
        m[:, :SPLIT], feat_lo_ref[...],
        precision=jax.lax.Precision.HIGHEST,
    )
    cnts_ref[...] = jnp.sum(m, axis=1, keepdims=True)


_tc_dense = pl.pallas_call(
    _tc_dense_body,
    out_shape=(
        jax.ShapeDtypeStruct((NB, CH), jnp.float32),
        jax.ShapeDtypeStruct((NB, 1), jnp.float32),
    ),
)


def _combine_body(sums_ref, cnts_ref, psum_ref, out_ref):
    total = sums_ref[...] + jnp.sum(psum_ref[...], axis=0)
    out_ref[...] = total / jnp.maximum(cnts_ref[...], 1.0)


_combine = pl.pallas_call(
    _combine_body,
    out_shape=jax.ShapeDtypeStruct((NB, CH), jnp.float32),
)


def kernel(features, coords, is_inside, bbox_sample_count, batch_splits):
    del coords, bbox_sample_count, batch_splits
    mask32 = is_inside.astype(jnp.int32)
    maskt_hi = is_inside[:, SPLIT:].T.astype(jnp.float32)
    psum = _sc_partial(features, maskt_hi)
    sums_lo, cnts = _tc_dense(mask32, features[:SPLIT])
    return _combine(sums_lo, cnts, psum)
